# Initial kernel scaffold; baseline (speedup 1.0000x reference)
import jax
import jax.numpy as jnp
from jax.experimental import pallas as pl


def kernel(x, edge_index, W_enc, b_enc, W_edge, W_node, W1, b1, W2, b2, W3, b3):
    raise NotImplementedError("write your pallas kernel here")


# speedup vs baseline: 5.8233x; 5.8233x over previous
import functools

import jax
import jax.numpy as jnp
from jax import lax
from jax.experimental import pallas as pl
from jax.experimental.pallas import tpu as pltpu
from jax.experimental.pallas import tpu_sc as plsc

NC = 2
NS = 16
CHUNK = 80


def _rnd(a):
    return a.astype(jnp.bfloat16).astype(jnp.float32)


def _dot(a, b):
    return jnp.dot(a, b, preferred_element_type=jnp.float32,
                   precision=lax.Precision.HIGHEST)


def _segment_sum_sc(hb, src3, dst3, zeros):
    N, D = hb.shape
    rows_per_tile = src3.shape[1]
    stripe = (N // NS) & ~7
    rem = N - stripe * NS

    mesh = plsc.VectorSubcoreMesh(core_axis_name="c", subcore_axis_name="s")

    @functools.partial(
        pl.kernel,
        out_type=jax.ShapeDtypeStruct((NC, N, D), jnp.float32),
        mesh=mesh,
        scratch_types=[
            pltpu.VMEM((rows_per_tile, CHUNK), jnp.int32),
            pltpu.VMEM((rows_per_tile, CHUNK), jnp.int32),
            pltpu.VMEM((CHUNK, D), jnp.float32),
            pltpu.VMEM_SHARED((N, D), jnp.float32),
            pltpu.SemaphoreType.DMA,
        ],
    )
    def segsum(h_hbm, src_hbm, dst_hbm, z_hbm, out_hbm,
               src_v, dst_v, rows_v, acc_sh, sem):
        cid = lax.axis_index("c")
        sid = lax.axis_index("s")
        wid = sid * NC + cid
        pltpu.sync_copy(z_hbm.at[pl.ds(sid * stripe, stripe)],
                        acc_sh.at[pl.ds(sid * stripe, stripe)])
        @pl.when(sid == 0)
        def _():
            pltpu.sync_copy(z_hbm.at[pl.ds(stripe * NS, rem)],
                            acc_sh.at[pl.ds(stripe * NS, rem)])
        pltpu.sync_copy(src_hbm.at[wid], src_v)
        pltpu.sync_copy(dst_hbm.at[wid], dst_v)
        plsc.subcore_barrier()

        def body(j, carry):
            pltpu.async_copy(h_hbm.at[src_v.at[j]], rows_v, sem).wait()
            pltpu.sync_copy(rows_v, acc_sh.at[dst_v.at[j]], add=True)
            return carry

        lax.fori_loop(0, rows_per_tile, body, 0)
        plsc.subcore_barrier()
        pltpu.sync_copy(acc_sh.at[pl.ds(sid * stripe, stripe)],
                        out_hbm.at[cid, pl.ds(sid * stripe, stripe)])
        @pl.when(sid == 0)
        def _():
            pltpu.sync_copy(acc_sh.at[pl.ds(stripe * NS, rem)],
                            out_hbm.at[cid, pl.ds(stripe * NS, rem)])

    return segsum(hb, src3, dst3, zeros)


def _encoder(x, W_enc, b_enc, blk):
    N, D = x.shape

    def body(x_ref, w_ref, b_ref, o_ref, ob_ref):
        h = jnp.maximum(_dot(_rnd(x_ref[...]), w_ref[...]) + b_ref[...], 0.0)
        o_ref[...] = h
        ob_ref[...] = _rnd(h)

    return pl.pallas_call(
        body,
        grid=(N // blk,),
        in_specs=[pl.BlockSpec((blk, D), lambda i: (i, 0)),
                  pl.BlockSpec((D, D), lambda i: (0, 0)),
                  pl.BlockSpec((1, D), lambda i: (0, 0))],
        out_specs=[pl.BlockSpec((blk, D), lambda i: (i, 0)),
                   pl.BlockSpec((blk, D), lambda i: (i, 0))],
        out_shape=[jax.ShapeDtypeStruct((N, D), jnp.float32),
                   jax.ShapeDtypeStruct((N, D), jnp.float32)],
    )(x, _rnd(W_enc), b_enc.reshape(1, D))


def _node_update(P, We, Wn, h, blk):
    N, D = h.shape

    def body(p_ref, we_ref, wn_ref, h_ref, o_ref, ob_ref):
        s = p_ref[0] + p_ref[1]
        agg = _dot(s, we_ref[...])
        hn = jnp.maximum(_dot(_rnd(agg), wn_ref[...]) + h_ref[...], 0.0)
        o_ref[...] = hn
        ob_ref[...] = _rnd(hn)

    return pl.pallas_call(
        body,
        grid=(N // blk,),
        in_specs=[pl.BlockSpec((2, blk, D), lambda i: (0, i, 0)),
                  pl.BlockSpec((D, D), lambda i: (0, 0)),
                  pl.BlockSpec((D, D), lambda i: (0, 0)),
                  pl.BlockSpec((blk, D), lambda i: (i, 0))],
        out_specs=[pl.BlockSpec((blk, D), lambda i: (i, 0)),
                   pl.BlockSpec((blk, D), lambda i: (i, 0))],
        out_shape=[jax.ShapeDtypeStruct((N, D), jnp.float32),
                   jax.ShapeDtypeStruct((N, D), jnp.float32)],
    )(P, _rnd(We), _rnd(Wn), h)


def _decoder(h, W1, b1, W2, b2, W3, b3, blk):
    N, D = h.shape
    D2, D4 = W1.shape[1], W2.shape[1]

    def body(h_ref, w1_ref, b1_ref, w2_ref, b2_ref, w3_ref, b3_ref, o_ref):
        t = jnp.maximum(_dot(_rnd(h_ref[...]), w1_ref[...]) + b1_ref[...], 0.0)
        t = jnp.maximum(_dot(_rnd(t), w2_ref[...]) + b2_ref[...], 0.0)
        o_ref[...] = _dot(_rnd(t), w3_ref[...]) + b3_ref[...]

    return pl.pallas_call(
        body,
        grid=(N // blk,),
        in_specs=[pl.BlockSpec((blk, D), lambda i: (i, 0)),
                  pl.BlockSpec((D, D2), lambda i: (0, 0)),
                  pl.BlockSpec((1, D2), lambda i: (0, 0)),
                  pl.BlockSpec((D2, D4), lambda i: (0, 0)),
                  pl.BlockSpec((1, D4), lambda i: (0, 0)),
                  pl.BlockSpec((D4, 1), lambda i: (0, 0)),
                  pl.BlockSpec((1, 1), lambda i: (0, 0))],
        out_specs=pl.BlockSpec((blk, 1), lambda i: (i, 0)),
        out_shape=jax.ShapeDtypeStruct((N, 1), jnp.float32),
    )(h, _rnd(W1), b1.reshape(1, D2), _rnd(W2), b2.reshape(1, D4),
      _rnd(W3), b3.reshape(1, 1))


def kernel(x, edge_index, W_enc, b_enc, W_edge, W_node, W1, b1, W2, b2, W3, b3):
    N, D = x.shape
    E = edge_index.shape[1]
    n_layers = W_edge.shape[0]
    blk = 1000

    rows_per_tile = E // (CHUNK * NC * NS)
    src3 = edge_index[0].reshape(NC * NS, rows_per_tile, CHUNK)
    dst3 = edge_index[1].reshape(NC * NS, rows_per_tile, CHUNK)
    zeros = jnp.zeros((N, D), jnp.float32)

    h, hb = _encoder(x, W_enc, b_enc, blk)
    for l in range(n_layers):
        P = _segment_sum_sc(hb, src3, dst3, zeros)
        h, hb = _node_update(P, W_edge[l], W_node[l], h, blk)
    return _decoder(h, W1, b1, W2, b2, W3, b3, blk)


# speedup vs baseline: 9.1382x; 1.5693x over previous
import functools

import jax
import jax.numpy as jnp
from jax import lax
from jax.experimental import pallas as pl
from jax.experimental.pallas import tpu as pltpu
from jax.experimental.pallas import tpu_sc as plsc

NC = 2
NS = 16
CHUNK = 50
GSZ = 4


def _rnd(a):
    return a.astype(jnp.bfloat16).astype(jnp.float32)


def _dot(a, b):
    return jnp.dot(a, b, preferred_element_type=jnp.float32,
                   precision=lax.Precision.HIGHEST)


def _segment_sum_sc(hb, src4, dst4, zeros):
    N, D = hb.shape
    ngroup, gsz, chunk = src4.shape[1], src4.shape[2], src4.shape[3]
    stripe = (N // NS) & ~7
    rem = N - stripe * NS

    mesh = plsc.VectorSubcoreMesh(core_axis_name="c", subcore_axis_name="s")

    @functools.partial(
        pl.kernel,
        out_type=jax.ShapeDtypeStruct((NC, N, D), jnp.float32),
        mesh=mesh,
        scratch_types=[
            pltpu.VMEM((2, gsz, chunk), jnp.int32),
            pltpu.VMEM((2, gsz, chunk), jnp.int32),
            pltpu.VMEM((gsz, chunk, D), jnp.float32),
            pltpu.VMEM_SHARED((N, D), jnp.float32),
            pltpu.SemaphoreType.DMA((gsz,)),
            pltpu.SemaphoreType.DMA((2,)),
            pltpu.SemaphoreType.DMA((2,)),
        ],
    )
    def segsum(h_hbm, src_hbm, dst_hbm, z_hbm, out_hbm,
               src_v, dst_v, rows_v, acc_sh, gsem, isem, jsem):
        cid = lax.axis_index("c")
        sid = lax.axis_index("s")
        wid = sid * NC + cid
        pltpu.sync_copy(z_hbm.at[pl.ds(sid * stripe, stripe)],
                        acc_sh.at[pl.ds(sid * stripe, stripe)])
        @pl.when(sid == 0)
        def _():
            pltpu.sync_copy(z_hbm.at[pl.ds(stripe * NS, rem)],
                            acc_sh.at[pl.ds(stripe * NS, rem)])
        pltpu.sync_copy(src_hbm.at[wid, 0], src_v.at[0])
        pltpu.sync_copy(dst_hbm.at[wid, 0], dst_v.at[0])
        plsc.subcore_barrier()

        def gather(p, b):
            pltpu.async_copy(h_hbm.at[src_v.at[p, b]], rows_v.at[b],
                             gsem.at[b])

        for b in range(gsz):
            gather(0, b)

        def body(g, carry):
            p = g % 2
            q = 1 - p
            @pl.when(g + 1 < ngroup)
            def _():
                pltpu.async_copy(src_hbm.at[wid, g + 1], src_v.at[q],
                                 isem.at[q])
                pltpu.async_copy(dst_hbm.at[wid, g + 1], dst_v.at[q],
                                 jsem.at[q])
            for b in range(gsz):
                pltpu.make_async_copy(h_hbm.at[src_v.at[0, 0]], rows_v.at[b],
                                      gsem.at[b]).wait()
                pltpu.sync_copy(rows_v.at[b], acc_sh.at[dst_v.at[p, b]],
                                add=True)
                @pl.when(g + 1 < ngroup)
                def _():
                    if b == 0:
                        pltpu.make_async_copy(src_hbm.at[0, 0], src_v.at[q],
                                              isem.at[q]).wait()
                        pltpu.make_async_copy(dst_hbm.at[0, 0], dst_v.at[q],
                                              jsem.at[q]).wait()
                    gather(q, b)
            return carry

        lax.fori_loop(0, ngroup, body, 0)
        plsc.subcore_barrier()
        pltpu.sync_copy(acc_sh.at[pl.ds(sid * stripe, stripe)],
                        out_hbm.at[cid, pl.ds(sid * stripe, stripe)])
        @pl.when(sid == 0)
        def _():
            pltpu.sync_copy(acc_sh.at[pl.ds(stripe * NS, rem)],
                            out_hbm.at[cid, pl.ds(stripe * NS, rem)])

    return segsum(hb, src4, dst4, zeros)


def _encoder(x, W_enc, b_enc, blk):
    N, D = x.shape

    def body(x_ref, w_ref, b_ref, o_ref, ob_ref):
        h = jnp.maximum(_dot(_rnd(x_ref[...]), w_ref[...]) + b_ref[...], 0.0)
        o_ref[...] = h
        ob_ref[...] = _rnd(h)

    return pl.pallas_call(
        body,
        grid=(N // blk,),
        in_specs=[pl.BlockSpec((blk, D), lambda i: (i, 0)),
                  pl.BlockSpec((D, D), lambda i: (0, 0)),
                  pl.BlockSpec((1, D), lambda i: (0, 0))],
        out_specs=[pl.BlockSpec((blk, D), lambda i: (i, 0)),
                   pl.BlockSpec((blk, D), lambda i: (i, 0))],
        out_shape=[jax.ShapeDtypeStruct((N, D), jnp.float32),
                   jax.ShapeDtypeStruct((N, D), jnp.float32)],
    )(x, _rnd(W_enc), b_enc.reshape(1, D))


def _node_update(P, We, Wn, h, blk):
    N, D = h.shape

    def body(p_ref, we_ref, wn_ref, h_ref, o_ref, ob_ref):
        s = p_ref[0] + p_ref[1]
        agg = _dot(s, we_ref[...])
        hn = jnp.maximum(_dot(_rnd(agg), wn_ref[...]) + h_ref[...], 0.0)
        o_ref[...] = hn
        ob_ref[...] = _rnd(hn)

    return pl.pallas_call(
        body,
        grid=(N // blk,),
        in_specs=[pl.BlockSpec((2, blk, D), lambda i: (0, i, 0)),
                  pl.BlockSpec((D, D), lambda i: (0, 0)),
                  pl.BlockSpec((D, D), lambda i: (0, 0)),
                  pl.BlockSpec((blk, D), lambda i: (i, 0))],
        out_specs=[pl.BlockSpec((blk, D), lambda i: (i, 0)),
                   pl.BlockSpec((blk, D), lambda i: (i, 0))],
        out_shape=[jax.ShapeDtypeStruct((N, D), jnp.float32),
                   jax.ShapeDtypeStruct((N, D), jnp.float32)],
    )(P, _rnd(We), _rnd(Wn), h)


def _decoder(h, W1, b1, W2, b2, W3, b3, blk):
    N, D = h.shape
    D2, D4 = W1.shape[1], W2.shape[1]

    def body(h_ref, w1_ref, b1_ref, w2_ref, b2_ref, w3_ref, b3_ref, o_ref):
        t = jnp.maximum(_dot(_rnd(h_ref[...]), w1_ref[...]) + b1_ref[...], 0.0)
        t = jnp.maximum(_dot(_rnd(t), w2_ref[...]) + b2_ref[...], 0.0)
        o_ref[...] = _dot(_rnd(t), w3_ref[...]) + b3_ref[...]

    return pl.pallas_call(
        body,
        grid=(N // blk,),
        in_specs=[pl.BlockSpec((blk, D), lambda i: (i, 0)),
                  pl.BlockSpec((D, D2), lambda i: (0, 0)),
                  pl.BlockSpec((1, D2), lambda i: (0, 0)),
                  pl.BlockSpec((D2, D4), lambda i: (0, 0)),
                  pl.BlockSpec((1, D4), lambda i: (0, 0)),
                  pl.BlockSpec((D4, 1), lambda i: (0, 0)),
                  pl.BlockSpec((1, 1), lambda i: (0, 0))],
        out_specs=pl.BlockSpec((blk, 1), lambda i: (i, 0)),
        out_shape=jax.ShapeDtypeStruct((N, 1), jnp.float32),
    )(h, _rnd(W1), b1.reshape(1, D2), _rnd(W2), b2.reshape(1, D4),
      _rnd(W3), b3.reshape(1, 1))


def kernel(x, edge_index, W_enc, b_enc, W_edge, W_node, W1, b1, W2, b2, W3, b3):
    N, D = x.shape
    E = edge_index.shape[1]
    n_layers = W_edge.shape[0]
    blk = 1000

    ngroup = E // (CHUNK * GSZ * NC * NS)
    src4 = edge_index[0].reshape(NC * NS, ngroup, GSZ, CHUNK)
    dst4 = edge_index[1].reshape(NC * NS, ngroup, GSZ, CHUNK)
    zeros = jnp.zeros((N, D), jnp.float32)

    h, hb = _encoder(x, W_enc, b_enc, blk)
    for l in range(n_layers):
        P = _segment_sum_sc(hb, src4, dst4, zeros)
        h, hb = _node_update(P, W_edge[l], W_node[l], h, blk)
    return _decoder(h, W1, b1, W2, b2, W3, b3, blk)
